# Initial kernel scaffold; baseline (speedup 1.0000x reference)
import jax
import jax.numpy as jnp
from jax.experimental import pallas as pl


def kernel(tasks, lookup_table):
    raise NotImplementedError("write your pallas kernel here")


# speedup vs baseline: 134.4587x; 134.4587x over previous
import functools

import jax
import jax.numpy as jnp
from jax import lax
from jax.experimental import pallas as pl
from jax.experimental.pallas import tpu as pltpu
from jax.experimental.pallas import tpu_sc as plsc

_NC = 2
_NS = 16
_L = 16
_NW = _NC * _NS


@functools.partial(jax.jit, static_argnums=(2, 3))
def _route(flat_tasks, lookup_table, n_per_w, chunk):
    n_total = flat_tasks.shape[0]
    table_n = lookup_table.shape[0]
    mesh = plsc.VectorSubcoreMesh(core_axis_name="c", subcore_axis_name="s")

    @functools.partial(
        pl.kernel,
        mesh=mesh,
        out_type=jax.ShapeDtypeStruct((n_total,), jnp.int32),
        scratch_types=[
            pltpu.VMEM((table_n,), jnp.int32),
            pltpu.VMEM((chunk,), jnp.int32),
        ],
        compiler_params=pltpu.CompilerParams(needs_layout_passes=False),
    )
    def k(tasks_hbm, table_hbm, out_hbm, table_v, idx_v):
        wid = lax.axis_index("s") * _NC + lax.axis_index("c")
        base = wid * n_per_w
        pltpu.sync_copy(table_hbm, table_v)

        def chunk_body(ci, carry):
            off = base + ci * chunk
            pltpu.sync_copy(tasks_hbm.at[pl.ds(off, chunk)], idx_v)

            def vec_body(vi, c2):
                raw = idx_v[pl.ds(vi * _L, _L)]
                clamped = jnp.clip(raw, 0, table_n - 1)
                vals = plsc.load_gather(table_v, [clamped])
                ok = (raw >= 0) & (raw < table_n)
                idx_v[pl.ds(vi * _L, _L)] = jnp.where(ok, vals, -1)
                return c2

            lax.fori_loop(0, chunk // _L, vec_body, 0)
            pltpu.sync_copy(idx_v, out_hbm.at[pl.ds(off, chunk)])
            return carry

        lax.fori_loop(0, n_per_w // chunk, chunk_body, 0)

    return k(flat_tasks, lookup_table)


def kernel(tasks, lookup_table):
    b, t = tasks.shape
    n_total = b * t
    n_per_w = n_total // _NW
    chunk = 12800
    assert n_per_w % chunk == 0 and chunk % _L == 0
    flat = tasks.reshape(n_total)
    out = _route(flat, lookup_table, n_per_w, chunk)
    return out.reshape(b, t)


# speedup vs baseline: 173.8564x; 1.2930x over previous
import functools

import jax
import jax.numpy as jnp
from jax import lax
from jax.experimental import pallas as pl
from jax.experimental.pallas import tpu as pltpu
from jax.experimental.pallas import tpu_sc as plsc

_NC = 2
_NS = 16
_L = 16
_NW = _NC * _NS


@functools.partial(jax.jit, static_argnums=(2, 3))
def _route(flat_tasks, lookup_table, n_per_w, chunk):
    n_total = flat_tasks.shape[0]
    table_n = lookup_table.shape[0]
    mesh = plsc.VectorSubcoreMesh(core_axis_name="c", subcore_axis_name="s")

    @functools.partial(
        pl.kernel,
        mesh=mesh,
        out_type=jax.ShapeDtypeStruct((n_total,), jnp.int32),
        scratch_types=[
            pltpu.VMEM((table_n,), jnp.int32),
            pltpu.VMEM((chunk,), jnp.int32),
            pltpu.VMEM((chunk,), jnp.int32),
        ],
        compiler_params=pltpu.CompilerParams(needs_layout_passes=False),
    )
    def k(tasks_hbm, table_hbm, out_hbm, table_v, idx_v, out_v):
        wid = lax.axis_index("s") * _NC + lax.axis_index("c")
        base = wid * n_per_w
        pltpu.sync_copy(table_hbm, table_v)

        def chunk_body(ci, carry):
            off = base + ci * chunk
            pltpu.sync_copy(tasks_hbm.at[pl.ds(off, chunk)], idx_v)

            @plsc.parallel_loop(0, chunk, _L, unroll=8)
            def vec_body(i):
                raw = idx_v[pl.ds(i, _L)]
                clamped = jnp.clip(raw, 0, table_n - 1)
                out_v[pl.ds(i, _L)] = plsc.load_gather(table_v, [clamped])

            pltpu.sync_copy(out_v, out_hbm.at[pl.ds(off, chunk)])
            return carry

        lax.fori_loop(0, n_per_w // chunk, chunk_body, 0)

    return k(flat_tasks, lookup_table)


def kernel(tasks, lookup_table):
    b, t = tasks.shape
    n_total = b * t
    n_per_w = n_total // _NW
    chunk = 12800
    assert n_per_w % chunk == 0 and chunk % _L == 0
    flat = tasks.reshape(n_total)
    out = _route(flat, lookup_table, n_per_w, chunk)
    return out.reshape(b, t)


# speedup vs baseline: 201.8376x; 1.1609x over previous
import functools

import jax
import jax.numpy as jnp
from jax import lax
from jax.experimental import pallas as pl
from jax.experimental.pallas import tpu as pltpu
from jax.experimental.pallas import tpu_sc as plsc

_NC = 2
_NS = 16
_L = 16
_NW = _NC * _NS


@functools.partial(jax.jit, static_argnums=(2,))
def _route(tasks, lookup_table, rows_per_chunk):
    b, t = tasks.shape
    table_n = lookup_table.shape[0]
    rows_per_w = b // _NW
    n_chunks = rows_per_w // rows_per_chunk
    col_starts = list(range(0, t - _L + 1, _L))
    if col_starts[-1] + _L < t:
        col_starts.append(t - _L)
    mesh = plsc.VectorSubcoreMesh(core_axis_name="c", subcore_axis_name="s")

    @functools.partial(
        pl.kernel,
        mesh=mesh,
        out_type=jax.ShapeDtypeStruct((b, t), jnp.int32),
        scratch_types=[
            pltpu.VMEM((table_n,), jnp.int32),
            pltpu.VMEM((rows_per_chunk, t), jnp.int32),
            pltpu.VMEM((rows_per_chunk, t), jnp.int32),
        ],
        compiler_params=pltpu.CompilerParams(needs_layout_passes=False),
    )
    def k(tasks_hbm, table_hbm, out_hbm, table_v, idx_v, out_v):
        wid = lax.axis_index("s") * _NC + lax.axis_index("c")
        base = wid * rows_per_w
        pltpu.sync_copy(table_hbm, table_v)

        def chunk_body(ci, carry):
            r0 = base + ci * rows_per_chunk
            pltpu.sync_copy(tasks_hbm.at[pl.ds(r0, rows_per_chunk), :], idx_v)

            @plsc.parallel_loop(0, rows_per_chunk, 1, unroll=2)
            def row_body(r):
                for j in col_starts:
                    raw = idx_v[r, pl.ds(j, _L)]
                    clamped = jnp.clip(raw, 0, table_n - 1)
                    out_v[r, pl.ds(j, _L)] = plsc.load_gather(
                        table_v, [clamped]
                    )

            pltpu.sync_copy(out_v, out_hbm.at[pl.ds(r0, rows_per_chunk), :])
            return carry

        lax.fori_loop(0, n_chunks, chunk_body, 0)

    return k(tasks, lookup_table)


def kernel(tasks, lookup_table):
    b, t = tasks.shape
    assert b % _NW == 0
    rows_per_w = b // _NW
    rows_per_chunk = 32
    assert rows_per_w % rows_per_chunk == 0
    return _route(tasks, lookup_table, rows_per_chunk)
